# baseline (speedup 1.0000x reference)
import functools

import jax
import jax.numpy as jnp
from jax import lax
from jax.experimental import pallas as pl
from jax.experimental.pallas import tpu as pltpu
from jax.experimental.pallas import tpu_sc as plsc

BATCH = 16384
EMBED_DIM = 32
HIDDEN_DIM = 64

_CHUNK = 128


def _make_sc_gather(batch, dim):
    info = plsc.get_sparse_core_info()
    nc, ns = info.num_cores, info.num_subcores
    nw = nc * ns
    b_per_w = batch // nw
    n_chunks = b_per_w // _CHUNK
    mesh = plsc.VectorSubcoreMesh(core_axis_name="c", subcore_axis_name="s")

    @functools.partial(
        pl.kernel,
        mesh=mesh,
        compiler_params=pltpu.CompilerParams(use_tc_tiling_on_sc=False),
        out_type=[
            jax.ShapeDtypeStruct((batch, dim), jnp.float32),
            jax.ShapeDtypeStruct((batch, dim), jnp.float32),
        ],
        scratch_types=[
            pltpu.VMEM((n_chunks, _CHUNK), jnp.int32),
            pltpu.VMEM((n_chunks, _CHUNK), jnp.int32),
            pltpu.VMEM((b_per_w, dim), jnp.float32),
            pltpu.VMEM((b_per_w, dim), jnp.float32),
            pltpu.SemaphoreType.DMA,
        ],
    )
    def gather(uidx_hbm, iidx_hbm, utab_hbm, itab_hbm, uout_hbm, iout_hbm,
               uidx_v, iidx_v, urows_v, irows_v, sem):
        wid = lax.axis_index("s") * nc + lax.axis_index("c")
        base = wid * b_per_w
        row0 = wid * n_chunks
        pltpu.sync_copy(uidx_hbm.at[pl.ds(row0, n_chunks)], uidx_v)
        pltpu.sync_copy(iidx_hbm.at[pl.ds(row0, n_chunks)], iidx_v)
        copies = []
        for j in range(n_chunks):
            copies.append(pltpu.async_copy(
                utab_hbm.at[uidx_v.at[j]],
                urows_v.at[pl.ds(j * _CHUNK, _CHUNK)], sem))
            copies.append(pltpu.async_copy(
                itab_hbm.at[iidx_v.at[j]],
                irows_v.at[pl.ds(j * _CHUNK, _CHUNK)], sem))
        for cp in copies:
            cp.wait()
        pltpu.sync_copy(urows_v, uout_hbm.at[pl.ds(base, b_per_w)])
        pltpu.sync_copy(irows_v, iout_hbm.at[pl.ds(base, b_per_w)])

    return gather


def _mlp_body(u_ref, i_ref, w1a_ref, w1b_ref, b1_ref, w2_ref, b2_ref, o_ref):
    h = (jnp.dot(u_ref[...], w1a_ref[...], preferred_element_type=jnp.float32)
         + jnp.dot(i_ref[...], w1b_ref[...], preferred_element_type=jnp.float32)
         + b1_ref[...])
    h = jnp.maximum(h, 0.0)
    o_ref[...] = jnp.sum(h * w2_ref[...], axis=1, keepdims=True) + b2_ref[...]


def _mlp(u_rows, i_rows, W1, b1, W2, b2):
    blk = 2048
    grid = BATCH // blk
    w1a = W1[:EMBED_DIM]
    w1b = W1[EMBED_DIM:]
    b1r = b1.reshape(1, HIDDEN_DIM)
    w2r = W2.reshape(1, HIDDEN_DIM)
    b2r = b2.reshape(1, 1)
    return pl.pallas_call(
        _mlp_body,
        grid=(grid,),
        in_specs=[
            pl.BlockSpec((blk, EMBED_DIM), lambda b: (b, 0)),
            pl.BlockSpec((blk, EMBED_DIM), lambda b: (b, 0)),
            pl.BlockSpec((EMBED_DIM, HIDDEN_DIM), lambda b: (0, 0)),
            pl.BlockSpec((EMBED_DIM, HIDDEN_DIM), lambda b: (0, 0)),
            pl.BlockSpec((1, HIDDEN_DIM), lambda b: (0, 0)),
            pl.BlockSpec((1, HIDDEN_DIM), lambda b: (0, 0)),
            pl.BlockSpec((1, 1), lambda b: (0, 0)),
        ],
        out_specs=pl.BlockSpec((blk, 1), lambda b: (b, 0)),
        out_shape=jax.ShapeDtypeStruct((BATCH, 1), jnp.float32),
    )(u_rows, i_rows, w1a, w1b, b1r, w2r, b2r)


def kernel(user_ids, item_ids, user_table, item_table, W1, b1, W2, b2):
    uids = user_ids.astype(jnp.int32).reshape(BATCH // _CHUNK, _CHUNK)
    iids = item_ids.astype(jnp.int32).reshape(BATCH // _CHUNK, _CHUNK)
    gather = _make_sc_gather(BATCH, EMBED_DIM)
    u_rows, i_rows = gather(uids, iids, user_table, item_table)
    return _mlp(u_rows, i_rows, W1, b1, W2, b2)
